# baseline (speedup 1.0000x reference)
import jax
import jax.numpy as jnp
from jax.experimental import pallas as pl
import jax.experimental.pallas.tpu as pltpu
import jax.experimental.pallas.tpu_sc as plsc

_H = 128
_NTOK = 3
_VTILE = 2048
_SC_WINDOW = 16


def _sc_gather(emb, idx):
    n = idx.shape[0]
    h = emb.shape[1]
    idx2 = idx.reshape(n // _SC_WINDOW, _SC_WINDOW)

    @pl.kernel(
        out_type=jax.ShapeDtypeStruct((n, h), emb.dtype),
        mesh=plsc.VectorSubcoreMesh(core_axis_name="c", subcore_axis_name="s"),
    )
    def gather_kernel(emb_hbm, i_hbm, o_hbm):
        def body(i_vmem, o_vmem):
            pltpu.sync_copy(emb_hbm.at[i_vmem.at[0]], o_vmem)

        pltpu.emit_pipeline(
            body,
            grid=(n // _SC_WINDOW,),
            in_specs=[pl.BlockSpec((1, _SC_WINDOW), lambda i: (i, 0))],
            out_specs=[pl.BlockSpec((_SC_WINDOW, h), lambda i: (i, 0))],
            core_axis_name=("c", "s"),
            dimension_semantics=(pltpu.PARALLEL,),
        )(i_hbm, o_hbm)

    return gather_kernel(emb, idx2)


def _fused_body(g_ref, w01_ref, w03_ref, w11_ref, w13_ref,
                outw_ref, outb_ref, out_ref, h2_ref):
    j = pl.program_id(0)

    @pl.when(j == 0)
    def _():
        nrows = g_ref.shape[0]
        b = nrows // _NTOK
        t0 = g_ref[0 * b:1 * b, :]
        t1 = g_ref[1 * b:2 * b, :]
        t2 = g_ref[2 * b:3 * b, :]

        def mm(a, w_ref):
            return jax.lax.dot(a, w_ref[...],
                               preferred_element_type=jnp.float32)

        a1 = jax.nn.relu((mm(t1, w01_ref) + mm(t2, w03_ref)) * 0.5)
        a0 = jax.nn.relu((mm(t0, w01_ref) + mm(t1, w03_ref)) * 0.5)
        h2_ref[...] = jax.nn.relu((mm(a0, w11_ref) + mm(a1, w13_ref)) * 0.5)

    out_ref[...] = jax.lax.dot(
        h2_ref[...], outw_ref[...],
        preferred_element_type=jnp.float32) + outb_ref[...]


def kernel(x, emb, rel_w, out_w, out_b, edge_src, edge_dst, edge_rel):
    B, L = x.shape
    V = out_w.shape[1]
    H = emb.shape[1]
    del edge_src, edge_dst, edge_rel

    idx = x[:, L - _NTOK:].T.reshape(-1)

    w01 = rel_w[0, 1]
    w03 = rel_w[0, 3]
    w11 = rel_w[1, 1]
    w13 = rel_w[1, 3]

    n_tiles = pl.cdiv(V, _VTILE)
    Vp = n_tiles * _VTILE
    outw_p = jnp.pad(out_w, ((0, 0), (0, Vp - V)))
    outb_p = jnp.pad(out_b, (0, Vp - V)).reshape(1, Vp)

    g = _sc_gather(emb, idx)

    pred_p = pl.pallas_call(
        _fused_body,
        grid=(n_tiles,),
        in_specs=[
            pl.BlockSpec((_NTOK * B, H), lambda j: (0, 0)),
            pl.BlockSpec((H, H), lambda j: (0, 0)),
            pl.BlockSpec((H, H), lambda j: (0, 0)),
            pl.BlockSpec((H, H), lambda j: (0, 0)),
            pl.BlockSpec((H, H), lambda j: (0, 0)),
            pl.BlockSpec((H, _VTILE), lambda j: (0, j)),
            pl.BlockSpec((1, _VTILE), lambda j: (0, j)),
        ],
        out_specs=pl.BlockSpec((B, _VTILE), lambda j: (0, j)),
        scratch_shapes=[pltpu.VMEM((B, H), jnp.float32)],
        out_shape=jax.ShapeDtypeStruct((B, Vp), jnp.float32),
    )(g, w01, w03, w11, w13, outw_p, outb_p)

    return pred_p[:, :V]


# speedup vs baseline: 1.3833x; 1.3833x over previous
import jax
import jax.numpy as jnp
from jax.experimental import pallas as pl
import jax.experimental.pallas.tpu as pltpu
import jax.experimental.pallas.tpu_sc as plsc

_H = 128
_NTOK = 3
_VTILE = 2048
_SC_WINDOW = 16


def _sc_gather(emb, idx):
    n = idx.shape[0]
    h = emb.shape[1]
    idx2 = idx.reshape(n // _SC_WINDOW, _SC_WINDOW)

    @pl.kernel(
        out_type=jax.ShapeDtypeStruct((n, h), emb.dtype),
        mesh=plsc.VectorSubcoreMesh(core_axis_name="c", subcore_axis_name="s"),
    )
    def gather_kernel(emb_hbm, i_hbm, o_hbm):
        def body(i_vmem, o_vmem):
            pltpu.sync_copy(emb_hbm.at[i_vmem.at[0]], o_vmem)

        pltpu.emit_pipeline(
            body,
            grid=(n // _SC_WINDOW,),
            in_specs=[pl.BlockSpec((1, _SC_WINDOW), lambda i: (i, 0))],
            out_specs=[pl.BlockSpec((_SC_WINDOW, h), lambda i: (i, 0))],
            core_axis_name=("c", "s"),
            dimension_semantics=(pltpu.PARALLEL,),
        )(i_hbm, o_hbm)

    return gather_kernel(emb, idx2)


def _fused_body(idx_ref, emb_ref, w01_ref, w03_ref, w11_ref, w13_ref,
                outw_ref, outb_ref, out_ref, g_ref, h2_ref, sem):
    j = pl.program_id(0)

    @pl.when(j == 0)
    def _():
        nrows = g_ref.shape[0]

        def issue(i, carry):
            r = idx_ref[i]
            pltpu.make_async_copy(
                emb_ref.at[pl.ds(r, 1), :], g_ref.at[pl.ds(i, 1), :], sem
            ).start()
            return carry

        jax.lax.fori_loop(0, nrows, issue, 0, unroll=8)

        def drain(i, carry):
            pltpu.make_async_copy(
                emb_ref.at[pl.ds(0, 1), :], g_ref.at[pl.ds(0, 1), :], sem
            ).wait()
            return carry

        jax.lax.fori_loop(0, nrows, drain, 0, unroll=8)

        b = nrows // _NTOK
        t0 = g_ref[0 * b:1 * b, :]
        t1 = g_ref[1 * b:2 * b, :]
        t2 = g_ref[2 * b:3 * b, :]

        def mm(a, w_ref):
            return jax.lax.dot(a, w_ref[...],
                               preferred_element_type=jnp.float32)

        a1 = jax.nn.relu((mm(t1, w01_ref) + mm(t2, w03_ref)) * 0.5)
        a0 = jax.nn.relu((mm(t0, w01_ref) + mm(t1, w03_ref)) * 0.5)
        h2_ref[...] = jax.nn.relu((mm(a0, w11_ref) + mm(a1, w13_ref)) * 0.5)

    out_ref[...] = jax.lax.dot(
        h2_ref[...], outw_ref[...],
        preferred_element_type=jnp.float32) + outb_ref[...]


def kernel(x, emb, rel_w, out_w, out_b, edge_src, edge_dst, edge_rel):
    B, L = x.shape
    V = out_w.shape[1]
    H = emb.shape[1]
    del edge_src, edge_dst, edge_rel

    idx = x[:, L - _NTOK:].T.reshape(-1)

    w01 = rel_w[0, 1]
    w03 = rel_w[0, 3]
    w11 = rel_w[1, 1]
    w13 = rel_w[1, 3]

    n_tiles = pl.cdiv(V, _VTILE)
    Vp = n_tiles * _VTILE
    outw_p = jnp.pad(out_w, ((0, 0), (0, Vp - V)))
    outb_p = jnp.pad(out_b, (0, Vp - V)).reshape(1, Vp)

    grid_spec = pltpu.PrefetchScalarGridSpec(
        num_scalar_prefetch=1,
        grid=(n_tiles,),
        in_specs=[
            pl.BlockSpec(memory_space=pltpu.MemorySpace.HBM),
            pl.BlockSpec((H, H), lambda j, *_: (0, 0)),
            pl.BlockSpec((H, H), lambda j, *_: (0, 0)),
            pl.BlockSpec((H, H), lambda j, *_: (0, 0)),
            pl.BlockSpec((H, H), lambda j, *_: (0, 0)),
            pl.BlockSpec((H, _VTILE), lambda j, *_: (0, j)),
            pl.BlockSpec((1, _VTILE), lambda j, *_: (0, j)),
        ],
        out_specs=pl.BlockSpec((B, _VTILE), lambda j, *_: (0, j)),
        scratch_shapes=[
            pltpu.VMEM((_NTOK * B, H), jnp.float32),
            pltpu.VMEM((B, H), jnp.float32),
            pltpu.SemaphoreType.DMA,
        ],
    )

    pred_p = pl.pallas_call(
        _fused_body,
        grid_spec=grid_spec,
        out_shape=jax.ShapeDtypeStruct((B, Vp), jnp.float32),
    )(idx, emb, w01, w03, w11, w13, outw_p, outb_p)

    return pred_p[:, :V]


# speedup vs baseline: 1.4446x; 1.0443x over previous
import jax
import jax.numpy as jnp
from jax.experimental import pallas as pl
import jax.experimental.pallas.tpu as pltpu
import jax.experimental.pallas.tpu_sc as plsc

_H = 128
_NTOK = 3
_VTILE = 2048
_SC_WINDOW = 16


def _sc_gather(emb, idx):
    n = idx.shape[0]
    h = emb.shape[1]
    idx2 = idx.reshape(n // _SC_WINDOW, _SC_WINDOW)

    @pl.kernel(
        out_type=jax.ShapeDtypeStruct((n, h), emb.dtype),
        mesh=plsc.VectorSubcoreMesh(core_axis_name="c", subcore_axis_name="s"),
    )
    def gather_kernel(emb_hbm, i_hbm, o_hbm):
        def body(i_vmem, o_vmem):
            pltpu.sync_copy(emb_hbm.at[i_vmem.at[0]], o_vmem)

        pltpu.emit_pipeline(
            body,
            grid=(n // _SC_WINDOW,),
            in_specs=[pl.BlockSpec((1, _SC_WINDOW), lambda i: (i, 0))],
            out_specs=[pl.BlockSpec((_SC_WINDOW, h), lambda i: (i, 0))],
            core_axis_name=("c", "s"),
            dimension_semantics=(pltpu.PARALLEL,),
        )(i_hbm, o_hbm)

    return gather_kernel(emb, idx2)


def _fused_body(idx_ref, emb_ref, w01_ref, w03_ref, w11_ref, w13_ref,
                outw_ref, outb_ref, out_ref, g_ref, h2_ref, sem):
    j = pl.program_id(0)

    @pl.when(j == 0)
    def _():
        nrows = g_ref.shape[0]

        def issue(i, carry):
            r = idx_ref[i]
            pltpu.make_async_copy(
                emb_ref.at[pl.ds(r, 1), :], g_ref.at[pl.ds(i, 1), :], sem
            ).start()
            return carry

        jax.lax.fori_loop(0, nrows, issue, 0, unroll=16)

        pltpu.make_async_copy(
            emb_ref.at[pl.ds(0, nrows), :], g_ref, sem
        ).wait()

        b = nrows // _NTOK
        t0 = g_ref[0 * b:1 * b, :]
        t1 = g_ref[1 * b:2 * b, :]
        t2 = g_ref[2 * b:3 * b, :]

        def mm(a, w_ref):
            return jax.lax.dot(a, w_ref[...],
                               preferred_element_type=jnp.float32)

        a1 = jax.nn.relu((mm(t1, w01_ref) + mm(t2, w03_ref)) * 0.5)
        a0 = jax.nn.relu((mm(t0, w01_ref) + mm(t1, w03_ref)) * 0.5)
        h2_ref[...] = jax.nn.relu((mm(a0, w11_ref) + mm(a1, w13_ref)) * 0.5)

    out_ref[...] = jax.lax.dot(
        h2_ref[...], outw_ref[...],
        preferred_element_type=jnp.float32) + outb_ref[...]


def kernel(x, emb, rel_w, out_w, out_b, edge_src, edge_dst, edge_rel):
    B, L = x.shape
    V = out_w.shape[1]
    H = emb.shape[1]
    del edge_src, edge_dst, edge_rel

    idx = x[:, L - _NTOK:].T.reshape(-1)

    w01 = rel_w[0, 1]
    w03 = rel_w[0, 3]
    w11 = rel_w[1, 1]
    w13 = rel_w[1, 3]

    n_tiles = pl.cdiv(V, _VTILE)
    outb2 = out_b.reshape(1, V)

    grid_spec = pltpu.PrefetchScalarGridSpec(
        num_scalar_prefetch=1,
        grid=(n_tiles,),
        in_specs=[
            pl.BlockSpec(memory_space=pltpu.MemorySpace.HBM),
            pl.BlockSpec((H, H), lambda j, *_: (0, 0)),
            pl.BlockSpec((H, H), lambda j, *_: (0, 0)),
            pl.BlockSpec((H, H), lambda j, *_: (0, 0)),
            pl.BlockSpec((H, H), lambda j, *_: (0, 0)),
            pl.BlockSpec((H, _VTILE), lambda j, *_: (0, j)),
            pl.BlockSpec((1, _VTILE), lambda j, *_: (0, j)),
        ],
        out_specs=pl.BlockSpec((B, _VTILE), lambda j, *_: (0, j)),
        scratch_shapes=[
            pltpu.VMEM((_NTOK * B, H), jnp.float32),
            pltpu.VMEM((B, H), jnp.float32),
            pltpu.SemaphoreType.DMA,
        ],
    )

    return pl.pallas_call(
        _fused_body,
        grid_spec=grid_spec,
        out_shape=jax.ShapeDtypeStruct((B, V), jnp.float32),
    )(idx, emb, w01, w03, w11, w13, out_w, outb2)


# speedup vs baseline: 1.5511x; 1.0738x over previous
import jax
import jax.numpy as jnp
from jax.experimental import pallas as pl
import jax.experimental.pallas.tpu as pltpu

_NTOK = 3
_VTILE = 2048


def _fused_body(idx_ref, emb_ref, relw_ref, outw_ref, outb_ref, out_ref,
                g_ref, w_ref, h2_ref, gsem, wsem):
    j = pl.program_id(0)

    @pl.when(j == 0)
    def _():
        nrows = g_ref.shape[0]

        pltpu.make_async_copy(relw_ref.at[0, 1], w_ref.at[0], wsem).start()
        pltpu.make_async_copy(relw_ref.at[0, 3], w_ref.at[1], wsem).start()
        pltpu.make_async_copy(relw_ref.at[1, 1], w_ref.at[2], wsem).start()
        pltpu.make_async_copy(relw_ref.at[1, 3], w_ref.at[3], wsem).start()

        def issue(i, carry):
            r = idx_ref[i]
            pltpu.make_async_copy(
                emb_ref.at[pl.ds(r, 1), :], g_ref.at[pl.ds(i, 1), :], gsem
            ).start()
            return carry

        jax.lax.fori_loop(0, nrows, issue, 0, unroll=16)

        pltpu.make_async_copy(relw_ref.at[0], w_ref, wsem).wait()
        pltpu.make_async_copy(
            emb_ref.at[pl.ds(0, nrows), :], g_ref, gsem
        ).wait()

        b = nrows // _NTOK
        t0 = g_ref[0 * b:1 * b, :]
        t1 = g_ref[1 * b:2 * b, :]
        t2 = g_ref[2 * b:3 * b, :]

        def mm(a, w):
            return jax.lax.dot(a, w, preferred_element_type=jnp.float32)

        w01 = w_ref[0]
        w03 = w_ref[1]
        a1 = jax.nn.relu((mm(t1, w01) + mm(t2, w03)) * 0.5)
        a0 = jax.nn.relu((mm(t0, w01) + mm(t1, w03)) * 0.5)
        h2_ref[...] = jax.nn.relu(
            (mm(a0, w_ref[2]) + mm(a1, w_ref[3])) * 0.5)

    out_ref[...] = jax.lax.dot(
        h2_ref[...], outw_ref[...],
        preferred_element_type=jnp.float32) + outb_ref[...]


def kernel(x, emb, rel_w, out_w, out_b, edge_src, edge_dst, edge_rel):
    B, L = x.shape
    V = out_w.shape[1]
    H = emb.shape[1]
    del edge_src, edge_dst, edge_rel

    idx = x[:, L - _NTOK:].T.reshape(-1)

    n_tiles = pl.cdiv(V, _VTILE)
    outb2 = out_b.reshape(1, V)

    grid_spec = pltpu.PrefetchScalarGridSpec(
        num_scalar_prefetch=1,
        grid=(n_tiles,),
        in_specs=[
            pl.BlockSpec(memory_space=pltpu.MemorySpace.HBM),
            pl.BlockSpec(memory_space=pltpu.MemorySpace.HBM),
            pl.BlockSpec((H, _VTILE), lambda j, *_: (0, j)),
            pl.BlockSpec((1, _VTILE), lambda j, *_: (0, j)),
        ],
        out_specs=pl.BlockSpec((B, _VTILE), lambda j, *_: (0, j)),
        scratch_shapes=[
            pltpu.VMEM((_NTOK * B, H), jnp.float32),
            pltpu.VMEM((4, H, H), jnp.float32),
            pltpu.VMEM((B, H), jnp.float32),
            pltpu.SemaphoreType.DMA,
            pltpu.SemaphoreType.DMA,
        ],
    )

    return pl.pallas_call(
        _fused_body,
        grid_spec=grid_spec,
        out_shape=jax.ShapeDtypeStruct((B, V), jnp.float32),
    )(idx, emb, rel_w, out_w, outb2)


# speedup vs baseline: 1.7159x; 1.1062x over previous
import jax
import jax.numpy as jnp
from jax.experimental import pallas as pl
import jax.experimental.pallas.tpu as pltpu

_NTOK = 3
_VTILE = 5120


def _fused_body(idx_ref, emb_ref, relw_ref, outw_ref, outb_ref, out_ref,
                g_ref, w_ref, h2_ref, gsem, wsem):
    j = pl.program_id(0)

    @pl.when(j == 0)
    def _():
        nrows = g_ref.shape[0]

        pltpu.make_async_copy(relw_ref.at[0, 1], w_ref.at[0], wsem).start()
        pltpu.make_async_copy(relw_ref.at[0, 3], w_ref.at[1], wsem).start()
        pltpu.make_async_copy(relw_ref.at[1, 1], w_ref.at[2], wsem).start()
        pltpu.make_async_copy(relw_ref.at[1, 3], w_ref.at[3], wsem).start()

        def issue(i, carry):
            r = idx_ref[i]
            pltpu.make_async_copy(
                emb_ref.at[pl.ds(r, 1), :], g_ref.at[pl.ds(i, 1), :], gsem
            ).start()
            return carry

        jax.lax.fori_loop(0, nrows, issue, 0, unroll=16)

        pltpu.make_async_copy(relw_ref.at[0], w_ref, wsem).wait()
        pltpu.make_async_copy(
            emb_ref.at[pl.ds(0, nrows), :], g_ref, gsem
        ).wait()

        b = nrows // _NTOK
        t0 = g_ref[0 * b:1 * b, :]
        t1 = g_ref[1 * b:2 * b, :]
        t2 = g_ref[2 * b:3 * b, :]

        def mm(a, w):
            return jax.lax.dot(a, w, preferred_element_type=jnp.float32)

        w01 = w_ref[0]
        w03 = w_ref[1]
        a1 = jax.nn.relu((mm(t1, w01) + mm(t2, w03)) * 0.5)
        a0 = jax.nn.relu((mm(t0, w01) + mm(t1, w03)) * 0.5)
        h2_ref[...] = jax.nn.relu(
            (mm(a0, w_ref[2]) + mm(a1, w_ref[3])) * 0.5)

    out_ref[...] = jax.lax.dot(
        h2_ref[...], outw_ref[...],
        preferred_element_type=jnp.float32) + outb_ref[...]


def kernel(x, emb, rel_w, out_w, out_b, edge_src, edge_dst, edge_rel):
    B, L = x.shape
    V = out_w.shape[1]
    H = emb.shape[1]
    del edge_src, edge_dst, edge_rel

    idx = x[:, L - _NTOK:].T.reshape(-1)

    n_tiles = pl.cdiv(V, _VTILE)
    outb2 = out_b.reshape(1, V)

    grid_spec = pltpu.PrefetchScalarGridSpec(
        num_scalar_prefetch=1,
        grid=(n_tiles,),
        in_specs=[
            pl.BlockSpec(memory_space=pltpu.MemorySpace.HBM),
            pl.BlockSpec(memory_space=pltpu.MemorySpace.HBM),
            pl.BlockSpec((H, _VTILE), lambda j, *_: (0, j)),
            pl.BlockSpec((1, _VTILE), lambda j, *_: (0, j)),
        ],
        out_specs=pl.BlockSpec((B, _VTILE), lambda j, *_: (0, j)),
        scratch_shapes=[
            pltpu.VMEM((_NTOK * B, H), jnp.float32),
            pltpu.VMEM((4, H, H), jnp.float32),
            pltpu.VMEM((B, H), jnp.float32),
            pltpu.SemaphoreType.DMA,
            pltpu.SemaphoreType.DMA,
        ],
    )

    return pl.pallas_call(
        _fused_body,
        grid_spec=grid_spec,
        out_shape=jax.ShapeDtypeStruct((B, V), jnp.float32),
    )(idx, emb, rel_w, out_w, outb2)
